# baseline (speedup 1.0000x reference)
import functools

import jax
import jax.numpy as jnp
from jax import lax
from jax.experimental import pallas as pl
from jax.experimental.pallas import tpu as pltpu
from jax.experimental.pallas import tpu_sc as plsc

_B = 16384
_L = 200
_DIM = 32
_DT = _DIM // 8
_BT = _B // 128

_NC = 2
_NS = 16
_NW = _NC * _NS
_TPW = _BT // _NW
_NBLK = _TPW * _L


def _gather_body(xt_hbm, table_hbm, out_hbm,
                 idx0, idx1, rows0, rows1, tiles0, tiles1,
                 semg0, semg1, semw0, semw1):
    wid = lax.axis_index("s") * _NC + lax.axis_index("c")
    idx_v = (idx0, idx1)
    rows_v = (rows0, rows1)
    tiles_v = (tiles0, tiles1)
    semg = (semg0, semg1)
    semw = (semw0, semw1)

    rowbase = [lax.iota(jnp.int32, 16) + 16 * g for g in range(8)]
    dsplat = [jnp.full((16,), d, jnp.int32) for d in range(_DIM)]

    def coords(i):
        g = lax.div(i, _L)
        l = lax.rem(i, _L)
        return l, wid * _TPW + g

    def fire(p, i):
        l, bt = coords(i)
        pltpu.sync_copy(xt_hbm.at[l, pl.ds(bt * 128, 128)], idx_v[p])
        pltpu.async_copy(table_hbm.at[idx_v[p]], rows_v[p], semg[p])

    def drain_gather(p):
        pltpu.make_async_copy(table_hbm.at[idx_v[p]], rows_v[p],
                              semg[p]).wait()

    def swizzle(p):
        for dt in range(_DT):
            for ds in range(8):
                d = dt * 8 + ds
                for g in range(8):
                    vals = plsc.load_gather(rows_v[p],
                                            [rowbase[g], dsplat[d]])
                    tiles_v[p][dt, ds, pl.ds(16 * g, 16)] = vals

    def fire_writes(p, i):
        l, bt = coords(i)
        for dt in range(_DT):
            pltpu.async_copy(tiles_v[p].at[dt], out_hbm.at[l, dt, bt],
                             semw[p])

    def drain_writes(p):
        for dt in range(_DT):
            pltpu.make_async_copy(tiles_v[p].at[dt], out_hbm.at[0, dt, 0],
                                  semw[p]).wait()

    fire(0, 0)
    fire(1, 1)

    def step(s, carry):
        for p in (0, 1):
            i = 2 * s + p

            @pl.when(s >= 1)
            def _(p=p):
                drain_writes(p)

            drain_gather(p)
            swizzle(p)
            fire_writes(p, i)

            @pl.when(i + 2 < _NBLK)
            def _(p=p, i=i):
                fire(p, i + 2)

        return carry

    lax.fori_loop(0, _NBLK // 2, step, 0)
    drain_writes(0)
    drain_writes(1)


@jax.jit
def _embedding_lookup(xt, table):
    mesh = plsc.VectorSubcoreMesh(core_axis_name="c", subcore_axis_name="s")
    return pl.kernel(
        _gather_body,
        mesh=mesh,
        out_type=jax.ShapeDtypeStruct((_L, _DT, _BT, 8, 128), jnp.float32),
        scratch_types=[
            pltpu.VMEM((128,), jnp.int32),
            pltpu.VMEM((128,), jnp.int32),
            pltpu.VMEM((128, _DIM), jnp.float32),
            pltpu.VMEM((128, _DIM), jnp.float32),
            pltpu.VMEM((_DT, 8, 128), jnp.float32),
            pltpu.VMEM((_DT, 8, 128), jnp.float32),
            pltpu.SemaphoreType.DMA,
            pltpu.SemaphoreType.DMA,
            pltpu.SemaphoreType.DMA,
            pltpu.SemaphoreType.DMA,
        ],
        compiler_params=pltpu.CompilerParams(use_tc_tiling_on_sc=False,
                                             needs_layout_passes=False),
    )(xt, table)


def kernel(x, table):
    xt = x.T.astype(jnp.int32)
    q = _embedding_lookup(xt, table)
    return q.transpose(2, 4, 0, 1, 3).reshape(_B, _L, _DIM)


# speedup vs baseline: 1.4933x; 1.4933x over previous
import functools

import jax
import jax.numpy as jnp
from jax import lax
from jax.experimental import pallas as pl
from jax.experimental.pallas import tpu as pltpu
from jax.experimental.pallas import tpu_sc as plsc

_B = 16384
_L = 200
_DIM = 32
_DT = _DIM // 8
_BT = _B // 128

_NC = 2
_NS = 16
_NW = _NC * _NS
_TPW = _BT // _NW
_NBLK = _TPW * _L


def _gather_body(xt_hbm, table_hbm, out_hbm,
                 idx0, idx1, rows0, rows1, tiles0, tiles1,
                 semi0, semi1, semg0, semg1, semw0, semw1):
    wid = lax.axis_index("s") * _NC + lax.axis_index("c")
    idx_v = (idx0, idx1)
    rows_v = (rows0, rows1)
    tiles_v = (tiles0, tiles1)
    semi = (semi0, semi1)
    semg = (semg0, semg1)
    semw = (semw0, semw1)

    rowbase = [lax.iota(jnp.int32, 16) + 16 * g for g in range(8)]
    dsplat = [jnp.full((16,), d, jnp.int32) for d in range(_DIM)]

    def coords(i):
        g = lax.div(i, _L)
        l = lax.rem(i, _L)
        return l, wid * _TPW + g

    def fire_idx(p, i):
        l, bt = coords(i)
        pltpu.async_copy(xt_hbm.at[l, pl.ds(bt * 128, 128)], idx_v[p],
                         semi[p])

    def drain_idx(p):
        pltpu.make_async_copy(xt_hbm.at[0, pl.ds(0, 128)], idx_v[p],
                              semi[p]).wait()

    def fire_gather(p):
        pltpu.async_copy(table_hbm.at[idx_v[p]], rows_v[p], semg[p])

    def drain_gather(p):
        pltpu.make_async_copy(table_hbm.at[idx_v[p]], rows_v[p],
                              semg[p]).wait()

    def swizzle(p):
        for dt in range(_DT):
            for ds in range(8):
                d = dt * 8 + ds
                vals = [plsc.load_gather(rows_v[p], [rowbase[g], dsplat[d]])
                        for g in range(8)]
                for g in range(8):
                    tiles_v[p][dt, ds, pl.ds(16 * g, 16)] = vals[g]

    def fire_writes(p, i):
        l, bt = coords(i)
        for dt in range(_DT):
            pltpu.async_copy(tiles_v[p].at[dt], out_hbm.at[l, dt, bt],
                             semw[p])

    def drain_writes(p):
        for dt in range(_DT):
            pltpu.make_async_copy(tiles_v[p].at[dt], out_hbm.at[0, dt, 0],
                                  semw[p]).wait()

    fire_idx(0, 0)
    drain_idx(0)
    fire_gather(0)
    fire_idx(1, 1)

    def step(s, carry):
        for p in (0, 1):
            i = 2 * s + p

            @pl.when(s >= 1)
            def _(p=p):
                drain_writes(p)

            drain_gather(p)

            @pl.when(i + 1 < _NBLK)
            def _(p=p):
                drain_idx(1 - p)
                fire_gather(1 - p)

            @pl.when(i + 2 < _NBLK)
            def _(p=p, i=i):
                fire_idx(p, i + 2)

            swizzle(p)
            fire_writes(p, i)

        return carry

    lax.fori_loop(0, _NBLK // 2, step, 0)
    drain_writes(0)
    drain_writes(1)


@jax.jit
def _embedding_lookup(xt, table):
    mesh = plsc.VectorSubcoreMesh(core_axis_name="c", subcore_axis_name="s")
    return pl.kernel(
        _gather_body,
        mesh=mesh,
        out_type=jax.ShapeDtypeStruct((_L, _DT, _BT, 8, 128), jnp.float32),
        scratch_types=[
            pltpu.VMEM((128,), jnp.int32),
            pltpu.VMEM((128,), jnp.int32),
            pltpu.VMEM((128, _DIM), jnp.float32),
            pltpu.VMEM((128, _DIM), jnp.float32),
            pltpu.VMEM((_DT, 8, 128), jnp.float32),
            pltpu.VMEM((_DT, 8, 128), jnp.float32),
            pltpu.SemaphoreType.DMA,
            pltpu.SemaphoreType.DMA,
            pltpu.SemaphoreType.DMA,
            pltpu.SemaphoreType.DMA,
            pltpu.SemaphoreType.DMA,
            pltpu.SemaphoreType.DMA,
        ],
        compiler_params=pltpu.CompilerParams(use_tc_tiling_on_sc=False,
                                             needs_layout_passes=False),
    )(xt, table)


def kernel(x, table):
    xt = x.T.astype(jnp.int32)
    q = _embedding_lookup(xt, table)
    return q.transpose(2, 4, 0, 1, 3).reshape(_B, _L, _DIM)
